# baseline (speedup 1.0000x reference)
import functools

import jax
import jax.numpy as jnp
from jax import lax
from jax.experimental import pallas as pl
from jax.experimental.pallas import tpu as pltpu
from jax.experimental.pallas import tpu_sc as plsc

B, L, C = 8, 512, 52
V, E, D = 128, 64, 256

NC, NS, LANES = 2, 16, 16
NW = NC * NS
NTOK = B * L
TPW = NTOK // NW
GRP = 2
RPG = GRP * C
NGRP = TPW // GRP
NCHUNK = D // LANES


def _prep_body(order_ref, slen_ref, ids_ref, idx_ref):
    m = pl.program_id(0)
    sl = slen_ref[m]
    ids = ids_ref[0]
    pos = jax.lax.broadcasted_iota(jnp.int32, (L, C), 0)
    coff = jax.lax.broadcasted_iota(jnp.int32, (L, C), 1) * V
    idx_ref[0] = jnp.where(pos < sl, ids, 0) + coff


def _prep(x_ids, order, slen):
    grid_spec = pltpu.PrefetchScalarGridSpec(
        num_scalar_prefetch=2,
        grid=(B,),
        in_specs=[
            pl.BlockSpec((1, L, C), lambda m, order_ref, slen_ref: (order_ref[m], 0, 0)),
        ],
        out_specs=pl.BlockSpec((1, L, C), lambda m, *_: (m, 0, 0)),
    )
    return pl.pallas_call(
        _prep_body,
        grid_spec=grid_spec,
        out_shape=jax.ShapeDtypeStruct((B, L, C), jnp.int32),
    )(order, slen, x_ids)


def _table_body(emb_ref, w_ref, b_ref, t_ref):
    c = pl.program_id(0)
    t = jax.lax.dot(emb_ref[...], w_ref[0], preferred_element_type=jnp.float32)
    bias = jnp.where(c == 0, b_ref[...], 0.0)
    t = t + bias
    lo = t[:, : D // 2].astype(jnp.bfloat16).astype(jnp.float32)
    hi = t[:, D // 2 :].astype(jnp.bfloat16).astype(jnp.float32)
    lo_bits = jax.lax.shift_right_logical(
        jax.lax.bitcast_convert_type(lo, jnp.int32), 16
    )
    hi_bits = jax.lax.bitwise_and(
        jax.lax.bitcast_convert_type(hi, jnp.int32), jnp.int32(-65536)
    )
    t_ref[0] = jax.lax.bitwise_or(lo_bits, hi_bits)


def _build_table(emb, w3, b2):
    return pl.pallas_call(
        _table_body,
        grid=(C,),
        in_specs=[
            pl.BlockSpec((V, E), lambda c: (0, 0)),
            pl.BlockSpec((1, E, D), lambda c: (c, 0, 0)),
            pl.BlockSpec((1, D), lambda c: (0, 0)),
        ],
        out_specs=pl.BlockSpec((1, V, D // 2), lambda c: (c, 0, 0)),
        out_shape=jax.ShapeDtypeStruct((C, V, D // 2), jnp.int32),
    )(emb, w3, b2)


NBUF = 4
HCHUNK = (D // 2) // LANES


def _sc_body(t_ref, idx_ref, out_ref, idx_v, rows_v, out_v, *sems):
    w = lax.axis_index("s") * NC + lax.axis_index("c")
    pltpu.sync_copy(idx_ref.at[w], idx_v)

    def start(gg, buf):
        pltpu.make_async_copy(t_ref.at[idx_v.at[gg]], rows_v.at[buf], sems[buf]).start()

    def process(gg, buf):
        pltpu.make_async_copy(t_ref.at[idx_v.at[gg]], rows_v.at[buf], sems[buf]).wait()
        for tok in range(GRP):
            base = tok * C

            def acc_body(r, carry):
                out = []
                for v in range(HCHUNK):
                    word = rows_v[buf, base + r, pl.ds(v * LANES, LANES)]
                    lo = jax.lax.bitcast_convert_type(
                        jax.lax.shift_left(word, 16), jnp.float32
                    )
                    hi = jax.lax.bitcast_convert_type(
                        jax.lax.bitwise_and(word, jnp.int32(-65536)), jnp.float32
                    )
                    out.append(carry[2 * v] + lo)
                    out.append(carry[2 * v + 1] + hi)
                return tuple(out)

            acc = tuple(jnp.zeros((LANES,), jnp.float32) for _ in range(2 * HCHUNK))
            row = gg * GRP + tok
            for v in range(HCHUNK):
                out_v[row, pl.ds(v * LANES, LANES)] = jnp.maximum(acc[2 * v], 0.0)
                out_v[row, pl.ds((HCHUNK + v) * LANES, LANES)] = jnp.maximum(
                    acc[2 * v + 1], 0.0
                )

    for buf in range(NBUF):
        start(buf, buf)

    def loop_body(g, _):
        for buf in range(NBUF):
            process(g + buf, buf)
            start(g + buf + NBUF, buf)
        return 0

    lax.fori_loop(0, (NGRP - NBUF) // NBUF, lambda i, c: loop_body(NBUF * i, c), 0)
    for buf in range(NBUF):
        process(NGRP - NBUF + buf, buf)
    pltpu.sync_copy(out_v, out_ref.at[pl.ds(w * TPW, TPW)])


def _sc_gather_sum(t_packed, idx3):
    mesh = plsc.VectorSubcoreMesh(core_axis_name="c", subcore_axis_name="s")
    f = functools.partial(
        pl.kernel,
        out_type=jax.ShapeDtypeStruct((NTOK, D), jnp.float32),
        mesh=mesh,
        scratch_types=[
            pltpu.VMEM((NGRP, RPG), jnp.int32),
            pltpu.VMEM((NBUF, RPG, D // 2), jnp.int32),
            pltpu.VMEM((TPW, D), jnp.float32),
        ]
        + [pltpu.SemaphoreType.DMA] * NBUF,
    )(_sc_body)
    return f(t_packed, idx3)


def kernel(x_ids, lengths, emb, W, b):
    order = jnp.argsort(-lengths, stable=True).astype(jnp.int32)
    sorted_len = lengths[order]
    reversed_indices = jnp.argsort(order, stable=True)

    idx = _prep(x_ids.astype(jnp.int32), order, sorted_len.astype(jnp.int32))
    t = _build_table(emb, W.reshape(C, E, D), b.reshape(1, D))
    y = _sc_gather_sum(t.reshape(C * V, D // 2), idx.reshape(NW, NGRP, RPG))
    return (y.reshape(B, L, D), sorted_len, reversed_indices)


# speedup vs baseline: 3.9699x; 3.9699x over previous
import functools

import jax
import jax.numpy as jnp
from jax.experimental import pallas as pl
from jax.experimental.pallas import tpu as pltpu

B, L, C = 8, 512, 52
V, E, D = 128, 64, 256
NPAIR = C // 2


def _table_body(emb_ref, w_ref, t_ref):
    t_ref[0] = jax.lax.dot(
        emb_ref[...], w_ref[0], preferred_element_type=jnp.float32
    ).astype(jnp.bfloat16)


def _build_table(emb, w3):
    return pl.pallas_call(
        _table_body,
        grid=(C,),
        in_specs=[
            pl.BlockSpec((V, E), lambda c: (0, 0)),
            pl.BlockSpec((1, E, D), lambda c: (c, 0, 0)),
        ],
        out_specs=pl.BlockSpec((1, V, D), lambda c: (c, 0, 0)),
        out_shape=jax.ShapeDtypeStruct((C, V, D), jnp.bfloat16),
    )(emb, w3)


def _fused_body(order_ref, slen_ref, ids_ref, t2_ref, b_ref, y_ref):
    m = pl.program_id(0)
    sl = slen_ref[m]
    ids = ids_ref[0]
    pos = jax.lax.broadcasted_iota(jnp.int32, (L, 1), 0)
    valid = pos < sl
    col = jax.lax.broadcasted_iota(jnp.int32, (L, 2 * V), 1)
    in_lo = col < V
    acc = jnp.full((L, D), 0.0, dtype=jnp.float32)
    for cc in range(NPAIR):
        id0 = jnp.where(valid, ids[:, 2 * cc : 2 * cc + 1], 0)
        id1 = jnp.where(valid, ids[:, 2 * cc + 1 : 2 * cc + 2], 0)
        sel = jnp.where(in_lo, id0, id1 + V)
        a2 = (col == sel).astype(jnp.bfloat16)
        acc += jax.lax.dot(a2, t2_ref[cc], preferred_element_type=jnp.float32)
    y_ref[0] = jax.nn.relu(acc + b_ref[...])


def _fused(x_ids, order, slen, t2, b2):
    grid_spec = pltpu.PrefetchScalarGridSpec(
        num_scalar_prefetch=2,
        grid=(B,),
        in_specs=[
            pl.BlockSpec((1, L, C), lambda m, order_ref, slen_ref: (order_ref[m], 0, 0)),
            pl.BlockSpec((NPAIR, 2 * V, D), lambda m, *_: (0, 0, 0)),
            pl.BlockSpec((1, D), lambda m, *_: (0, 0)),
        ],
        out_specs=pl.BlockSpec((1, L, D), lambda m, *_: (m, 0, 0)),
    )
    return pl.pallas_call(
        _fused_body,
        grid_spec=grid_spec,
        out_shape=jax.ShapeDtypeStruct((B, L, D), jnp.float32),
    )(order, slen, x_ids, t2, b2)


@functools.partial(jax.jit, static_argnames=())
def kernel(x_ids, lengths, emb, W, b):
    order = jnp.argsort(-lengths, stable=True).astype(jnp.int32)
    sorted_len = lengths[order]
    reversed_indices = jnp.argsort(order, stable=True)

    t = _build_table(emb, W.reshape(C, E, D))
    t2 = t.reshape(NPAIR, 2 * V, D)
    y = _fused(
        x_ids.astype(jnp.int32),
        order,
        sorted_len.astype(jnp.int32),
        t2,
        b.reshape(1, D),
    )
    return (y, sorted_len, reversed_indices)


# speedup vs baseline: 5.5796x; 1.4055x over previous
import jax
import jax.numpy as jnp
from jax.experimental import pallas as pl
from jax.experimental.pallas import tpu as pltpu

B, L, C = 8, 512, 52
V, E, D = 128, 64, 256
NPAIR = C // 2
K = C * V


def _table_body(emb_ref, w_ref, t_ref):
    t_ref[:V] = jax.lax.dot(
        emb_ref[...], w_ref[0, 0], preferred_element_type=jnp.float32
    ).astype(jnp.bfloat16)
    t_ref[V:] = jax.lax.dot(
        emb_ref[...], w_ref[0, 1], preferred_element_type=jnp.float32
    ).astype(jnp.bfloat16)


def _build_table(emb, w4):
    return pl.pallas_call(
        _table_body,
        grid=(NPAIR,),
        in_specs=[
            pl.BlockSpec((V, E), lambda c: (0, 0)),
            pl.BlockSpec((1, 2, E, D), lambda c: (c, 0, 0, 0)),
        ],
        out_specs=pl.BlockSpec((2 * V, D), lambda c: (c, 0)),
        out_shape=jax.ShapeDtypeStruct((K, D), jnp.bfloat16),
    )(emb, w4)


def _fused_body(order_ref, slen_ref, ids_ref, t_ref, b_ref, y_ref, a_ref):
    m = pl.program_id(0)
    sl = slen_ref[m]
    ids = ids_ref[0]
    pos = jax.lax.broadcasted_iota(jnp.int32, (L, 1), 0)
    valid = pos < sl
    ids_bf = jnp.where(valid, ids, 0).astype(jnp.bfloat16)
    colh = jax.lax.broadcasted_iota(jnp.int32, (L, V), 1).astype(jnp.bfloat16)
    one = jnp.bfloat16(1.0)
    zero = jnp.bfloat16(0.0)
    for cc in range(NPAIR):
        lo = jnp.where(colh == ids_bf[:, 2 * cc : 2 * cc + 1], one, zero)
        hi = jnp.where(colh == ids_bf[:, 2 * cc + 1 : 2 * cc + 2], one, zero)
        a_ref[:, pl.ds(cc * 2 * V, V)] = lo
        a_ref[:, pl.ds(cc * 2 * V + V, V)] = hi
    y = jax.lax.dot(a_ref[...], t_ref[...], preferred_element_type=jnp.float32)
    y_ref[0] = jax.nn.relu(y + b_ref[...])


def _fused(x_ids, order, slen, t, b2):
    grid_spec = pltpu.PrefetchScalarGridSpec(
        num_scalar_prefetch=2,
        grid=(B,),
        in_specs=[
            pl.BlockSpec((1, L, C), lambda m, order_ref, slen_ref: (order_ref[m], 0, 0)),
            pl.BlockSpec((K, D), lambda m, *_: (0, 0)),
            pl.BlockSpec((1, D), lambda m, *_: (0, 0)),
        ],
        out_specs=pl.BlockSpec((1, L, D), lambda m, *_: (m, 0, 0)),
        scratch_shapes=[pltpu.VMEM((L, K), jnp.bfloat16)],
    )
    return pl.pallas_call(
        _fused_body,
        grid_spec=grid_spec,
        out_shape=jax.ShapeDtypeStruct((B, L, D), jnp.float32),
    )(order, slen, x_ids, t, b2)


def kernel(x_ids, lengths, emb, W, b):
    order = jnp.argsort(-lengths, stable=True).astype(jnp.int32)
    sorted_len = lengths[order]
    reversed_indices = jnp.argsort(order, stable=True)

    t = _build_table(emb, W.reshape(NPAIR, 2, E, D))
    y = _fused(x_ids, order, sorted_len, t, b.reshape(1, D))
    return (y, sorted_len, reversed_indices)


# speedup vs baseline: 5.7311x; 1.0272x over previous
import jax
import jax.numpy as jnp
from jax.experimental import pallas as pl
from jax.experimental.pallas import tpu as pltpu

B, L, C = 8, 512, 52
V, E, D = 128, 64, 256
NPAIR = C // 2
K = C * V


def _sort_body(l_ref, order_ref, slen_ref, rev_ref):
    l = l_ref[...]
    l8 = jnp.broadcast_to(l, (B, B))
    lT = l8.T
    ri = jax.lax.broadcasted_iota(jnp.int32, (B, B), 0)
    ci = jax.lax.broadcasted_iota(jnp.int32, (B, B), 1)
    before = (l8 > lT) | ((l8 == lT) & (ci < ri))
    rank = jnp.sum(before.astype(jnp.int32), axis=1, keepdims=True)
    rev_ref[...] = rank.T
    eqm = jnp.broadcast_to(rank, (B, B)) == ci
    order_ref[...] = jnp.sum(jnp.where(eqm, ri, 0), axis=0, keepdims=True)
    slen_ref[...] = jnp.sum(jnp.where(eqm, lT, 0), axis=0, keepdims=True)


def _sort_lengths(lengths):
    shp = jax.ShapeDtypeStruct((1, B), jnp.int32)
    order, slen, rev = pl.pallas_call(
        _sort_body,
        out_shape=(shp, shp, shp),
    )(lengths.reshape(1, B))
    return order.reshape(B), slen.reshape(B), rev.reshape(B)


def _table_body(emb_ref, w_ref, b_ref, t_ref):
    cc = pl.program_id(0)
    t0 = jax.lax.dot(emb_ref[...], w_ref[:E], preferred_element_type=jnp.float32)
    t0 = t0 + jnp.where(cc == 0, b_ref[...], 0.0)
    t_ref[:V] = t0.astype(jnp.bfloat16)
    t_ref[V:] = jax.lax.dot(
        emb_ref[...], w_ref[E:], preferred_element_type=jnp.float32
    ).astype(jnp.bfloat16)


def _build_table(emb, W, b2):
    return pl.pallas_call(
        _table_body,
        grid=(NPAIR,),
        in_specs=[
            pl.BlockSpec((V, E), lambda c: (0, 0)),
            pl.BlockSpec((2 * E, D), lambda c: (c, 0)),
            pl.BlockSpec((1, D), lambda c: (0, 0)),
        ],
        out_specs=pl.BlockSpec((2 * V, D), lambda c: (c, 0)),
        out_shape=jax.ShapeDtypeStruct((K, D), jnp.bfloat16),
    )(emb, W, b2)


def _fused_body(order_ref, slen_ref, ids_ref, t_ref, y_ref, a_ref):
    m = pl.program_id(0)
    sl = slen_ref[m]
    ids = ids_ref[0]
    pos = jax.lax.broadcasted_iota(jnp.int32, (L, 1), 0)
    valid = pos < sl
    ids_bf = jnp.where(valid, ids, 0).astype(jnp.bfloat16)
    colh = jax.lax.broadcasted_iota(jnp.int32, (L, V), 1).astype(jnp.bfloat16)
    one = jnp.bfloat16(1.0)
    zero = jnp.bfloat16(0.0)
    for cc in range(NPAIR):
        lo = jnp.where(colh == ids_bf[:, 2 * cc : 2 * cc + 1], one, zero)
        hi = jnp.where(colh == ids_bf[:, 2 * cc + 1 : 2 * cc + 2], one, zero)
        a_ref[:, pl.ds(cc * 2 * V, V)] = lo
        a_ref[:, pl.ds(cc * 2 * V + V, V)] = hi
    y = jax.lax.dot(a_ref[...], t_ref[...], preferred_element_type=jnp.float32)
    y_ref[0] = jax.nn.relu(y)


def _fused(x_ids, order, slen, t):
    grid_spec = pltpu.PrefetchScalarGridSpec(
        num_scalar_prefetch=2,
        grid=(B,),
        in_specs=[
            pl.BlockSpec((1, L, C), lambda m, order_ref, slen_ref: (order_ref[m], 0, 0)),
            pl.BlockSpec((K, D), lambda m, *_: (0, 0)),
        ],
        out_specs=pl.BlockSpec((1, L, D), lambda m, *_: (m, 0, 0)),
        scratch_shapes=[pltpu.VMEM((L, K), jnp.bfloat16)],
    )
    return pl.pallas_call(
        _fused_body,
        grid_spec=grid_spec,
        out_shape=jax.ShapeDtypeStruct((B, L, D), jnp.float32),
    )(order, slen, x_ids, t)


def kernel(x_ids, lengths, emb, W, b):
    order, sorted_len, reversed_indices = _sort_lengths(lengths)
    t = _build_table(emb, W, b.reshape(1, D))
    y = _fused(x_ids, order, sorted_len, t)
    return (y, sorted_len, reversed_indices)


# speedup vs baseline: 8.3388x; 1.4550x over previous
import jax
import jax.numpy as jnp
from jax.experimental import pallas as pl
from jax.experimental.pallas import tpu as pltpu

B, L, C = 8, 512, 52
V, E, D = 128, 64, 256
NPAIR = C // 2
K = C * V


def _sort_body(l_ref, slen_ref, rev_ref):
    l = l_ref[...]
    l8 = jnp.broadcast_to(l, (B, B))
    lT = l8.T
    ri = jax.lax.broadcasted_iota(jnp.int32, (B, B), 0)
    ci = jax.lax.broadcasted_iota(jnp.int32, (B, B), 1)
    before = (l8 > lT) | ((l8 == lT) & (ci < ri))
    rank = jnp.sum(before.astype(jnp.int32), axis=1, keepdims=True)
    rev_ref[...] = rank.T
    eqm = jnp.broadcast_to(rank, (B, B)) == ci
    slen_ref[...] = jnp.sum(jnp.where(eqm, lT, 0), axis=0, keepdims=True)


def _sort_lengths(lengths):
    shp = jax.ShapeDtypeStruct((1, B), jnp.int32)
    slen, rev = pl.pallas_call(
        _sort_body,
        out_shape=(shp, shp),
    )(lengths.reshape(1, B))
    return slen.reshape(B), rev.reshape(B)


def _fused_body(rank_ref, len_ref, ids_ref, emb_ref, w_ref, b_ref, y_ref,
                t_ref, a_ref):
    m = pl.program_id(0)

    @pl.when(m == 0)
    def _build_table():
        for cc in range(NPAIR):
            t0 = jax.lax.dot(
                emb_ref[...], w_ref[pl.ds(2 * cc * E, E)],
                preferred_element_type=jnp.float32,
            )
            if cc == 0:
                t0 = t0 + b_ref[...]
            t_ref[pl.ds(cc * 2 * V, V)] = t0.astype(jnp.bfloat16)
            t_ref[pl.ds(cc * 2 * V + V, V)] = jax.lax.dot(
                emb_ref[...], w_ref[pl.ds((2 * cc + 1) * E, E)],
                preferred_element_type=jnp.float32,
            ).astype(jnp.bfloat16)

    sl = len_ref[m]
    ids = ids_ref[0]
    pos = jax.lax.broadcasted_iota(jnp.int32, (L, 1), 0)
    valid = pos < sl
    ids_bf = jnp.where(valid, ids, 0).astype(jnp.bfloat16)
    colh = jax.lax.broadcasted_iota(jnp.int32, (L, V), 1).astype(jnp.bfloat16)
    one = jnp.bfloat16(1.0)
    zero = jnp.bfloat16(0.0)
    for cc in range(NPAIR):
        lo = jnp.where(colh == ids_bf[:, 2 * cc : 2 * cc + 1], one, zero)
        hi = jnp.where(colh == ids_bf[:, 2 * cc + 1 : 2 * cc + 2], one, zero)
        a_ref[:, pl.ds(cc * 2 * V, V)] = lo
        a_ref[:, pl.ds(cc * 2 * V + V, V)] = hi
    y = jax.lax.dot(a_ref[...], t_ref[...], preferred_element_type=jnp.float32)
    y_ref[0] = jax.nn.relu(y)


def _fused(x_ids, rank, lengths, emb, W, b2):
    grid_spec = pltpu.PrefetchScalarGridSpec(
        num_scalar_prefetch=2,
        grid=(B,),
        in_specs=[
            pl.BlockSpec((1, L, C), lambda m, *_: (m, 0, 0)),
            pl.BlockSpec((V, E), lambda m, *_: (0, 0)),
            pl.BlockSpec((C * E, D), lambda m, *_: (0, 0)),
            pl.BlockSpec((1, D), lambda m, *_: (0, 0)),
        ],
        out_specs=pl.BlockSpec(
            (1, L, D), lambda m, rank_ref, len_ref: (rank_ref[m], 0, 0)
        ),
        scratch_shapes=[
            pltpu.VMEM((K, D), jnp.bfloat16),
            pltpu.VMEM((L, K), jnp.bfloat16),
        ],
    )
    return pl.pallas_call(
        _fused_body,
        grid_spec=grid_spec,
        out_shape=jax.ShapeDtypeStruct((B, L, D), jnp.float32),
    )(rank, lengths, x_ids, emb, W, b2)


def kernel(x_ids, lengths, emb, W, b):
    sorted_len, reversed_indices = _sort_lengths(lengths)
    y = _fused(x_ids, reversed_indices, lengths, emb, W, b.reshape(1, D))
    return (y, sorted_len, reversed_indices)


# speedup vs baseline: 8.4281x; 1.0107x over previous
import jax
import jax.numpy as jnp
from jax.experimental import pallas as pl
from jax.experimental.pallas import tpu as pltpu

B, L, C = 8, 512, 52
V, E, D = 128, 64, 256
NPAIR = C // 2
K = C * V


def _fused_body(len_ref, ids_ref, emb_ref, w_ref, b_ref, lv_ref, y_ref,
                slen_ref, rev_ref, t_ref, a_ref):
    m = pl.program_id(0)

    @pl.when(m == 0)
    def _build_table():
        for cc in range(NPAIR):
            t0 = jax.lax.dot(
                emb_ref[...], w_ref[pl.ds(2 * cc * E, E)],
                preferred_element_type=jnp.float32,
            )
            if cc == 0:
                t0 = t0 + b_ref[...]
            t_ref[pl.ds(cc * 2 * V, V)] = t0.astype(jnp.bfloat16)
            t_ref[pl.ds(cc * 2 * V + V, V)] = jax.lax.dot(
                emb_ref[...], w_ref[pl.ds((2 * cc + 1) * E, E)],
                preferred_element_type=jnp.float32,
            ).astype(jnp.bfloat16)

    l = lv_ref[...]
    l8 = jnp.broadcast_to(l, (B, B))
    lT = l8.T
    ri = jax.lax.broadcasted_iota(jnp.int32, (B, B), 0)
    ci = jax.lax.broadcasted_iota(jnp.int32, (B, B), 1)
    before = (l8 > lT) | ((l8 == lT) & (ci < ri))
    rank = jnp.sum(before.astype(jnp.int32), axis=1, keepdims=True)
    rev_ref[...] = rank.T
    eqm = jnp.broadcast_to(rank, (B, B)) == ci
    slen_ref[...] = jnp.sum(jnp.where(eqm, lT, 0), axis=0, keepdims=True)

    sl = len_ref[0, m]
    ids = ids_ref[0]
    pos = jax.lax.broadcasted_iota(jnp.int32, (L, 1), 0)
    valid = pos < sl
    ids_bf = jnp.where(valid, ids, 0).astype(jnp.bfloat16)
    colh = jax.lax.broadcasted_iota(jnp.int32, (L, V), 1).astype(jnp.bfloat16)
    one = jnp.bfloat16(1.0)
    zero = jnp.bfloat16(0.0)
    for cc in range(NPAIR):
        lo = jnp.where(colh == ids_bf[:, 2 * cc : 2 * cc + 1], one, zero)
        hi = jnp.where(colh == ids_bf[:, 2 * cc + 1 : 2 * cc + 2], one, zero)
        a_ref[:, pl.ds(cc * 2 * V, V)] = lo
        a_ref[:, pl.ds(cc * 2 * V + V, V)] = hi
    y = jax.lax.dot(a_ref[...], t_ref[...], preferred_element_type=jnp.float32)
    y_ref[0] = jax.nn.relu(y)


def _rank_of(m, len_ref):
    lm = len_ref[0, m]
    r = 0
    for j in range(B):
        lj = len_ref[0, j]
        gt = lj > lm
        tie = (lj == lm) & (j < m)
        r = r + jnp.where(gt | tie, 1, 0)
    return r


def _fused(x_ids, lengths2, emb, W, b2):
    grid_spec = pltpu.PrefetchScalarGridSpec(
        num_scalar_prefetch=1,
        grid=(B,),
        in_specs=[
            pl.BlockSpec((1, L, C), lambda m, *_: (m, 0, 0)),
            pl.BlockSpec((V, E), lambda m, *_: (0, 0)),
            pl.BlockSpec((C * E, D), lambda m, *_: (0, 0)),
            pl.BlockSpec((1, D), lambda m, *_: (0, 0)),
            pl.BlockSpec((1, B), lambda m, *_: (0, 0)),
        ],
        out_specs=(
            pl.BlockSpec((1, L, D), lambda m, len_ref: (_rank_of(m, len_ref), 0, 0)),
            pl.BlockSpec((1, B), lambda m, *_: (0, 0)),
            pl.BlockSpec((1, B), lambda m, *_: (0, 0)),
        ),
        scratch_shapes=[
            pltpu.VMEM((K, D), jnp.bfloat16),
            pltpu.VMEM((L, K), jnp.bfloat16),
        ],
    )
    return pl.pallas_call(
        _fused_body,
        grid_spec=grid_spec,
        out_shape=(
            jax.ShapeDtypeStruct((B, L, D), jnp.float32),
            jax.ShapeDtypeStruct((1, B), jnp.int32),
            jax.ShapeDtypeStruct((1, B), jnp.int32),
        ),
    )(lengths2, x_ids, emb, W, b2, lengths2)


def kernel(x_ids, lengths, emb, W, b):
    y, slen, rev = _fused(
        x_ids, lengths.reshape(1, B), emb, W, b.reshape(1, D)
    )
    return (y, slen.reshape(B), rev.reshape(B))
